# baseline (speedup 1.0000x reference)
import jax
import jax.numpy as jnp
from jax import lax
from jax.experimental import pallas as pl
from jax.experimental.pallas import tpu as pltpu
from jax.experimental.pallas import tpu_sc as plsc

F32 = jnp.float32
I32 = jnp.int32

_V = 1040000
_B = 16384
_F = 26
_D = 16
_NC = 2
_NS = 16
_NW = _NC * _NS
_PER_W = _B // _NW
_SUB = 128
_GPF = _PER_W // _SUB
_BLK = _PER_W // 16
_NBUF = 3


def _sc_body(xt_hbm, emb_hbm, thr_hbm, out_hbm,
             idx_v, rows_v, trans_v, thr_v, gsems, osems):
    wid = lax.axis_index("s") * _NC + lax.axis_index("c")
    wb = wid * _PER_W
    tile0 = wid * _GPF
    pltpu.sync_copy(thr_hbm, thr_v)
    lanes = lax.iota(I32, 16)
    c15 = jnp.full((16,), 15, I32)

    def fire_gathers(f):
        s = f % _NBUF
        pltpu.sync_copy(xt_hbm.at[f, pl.ds(tile0, _GPF)], idx_v.at[s])
        return [
            pltpu.async_copy(
                emb_hbm.at[idx_v.at[s, j]],
                rows_v.at[s, pl.ds(j * _SUB, _SUB)],
                gsems[s],
            )
            for j in range(_GPF)
        ]

    pending_g = {0: fire_gathers(0)}
    pending_o = {}

    for f in range(_F):
        s = f % _NBUF
        if f + 1 < _F:
            if f - 2 >= 0:
                for cp in pending_o.pop(f - 2):
                    cp.wait()
            pending_g[f + 1] = fire_gathers(f + 1)
        for cp in pending_g.pop(f):
            cp.wait()

        rows = rows_v.at[s]
        trans = trans_v.at[s]
        thr = plsc.load_gather(thr_v, [jnp.full((16,), f, I32)])

        def blk(bi, _, rows=rows, trans=trans, thr=thr):
            rb = bi * 16
            ridx = rb + lanes
            tclv = jnp.zeros((16,), I32) + lax.shift_right_logical(rb, 7)
            kbase = lax.bitwise_and(rb, 127) + lanes
            acc = jnp.zeros((16,), F32)
            diags = []
            for d in range(_D):
                cidx = lax.bitwise_and(lanes + d, c15)
                v = plsc.load_gather(rows, [ridx, cidx])
                diags.append((cidx, v))
                acc = acc + jnp.abs(v)
            m = jnp.where(acc - thr > 0, jnp.float32(1.0), jnp.float32(0.0))
            for cidx, v in diags:
                trv = lax.shift_right_logical(cidx, 3)
                kv = lax.bitwise_and(cidx, jnp.full((16,), 7, I32)) * 128
                plsc.store_scatter(trans, [trv, tclv, kv + kbase], v * m)
            return 0

        lax.fori_loop(0, _BLK, blk, 0)
        pending_o[f] = [
            pltpu.async_copy(
                trans.at[tr2],
                out_hbm.at[f, tr2, pl.ds(_GPF * wid, _GPF)],
                osems[s],
            )
            for tr2 in range(2)
        ]

    for f in sorted(pending_o):
        for cp in pending_o[f]:
            cp.wait()


_TC = _V // 128
_TPW = 254


_G = 4
_SPW = 66
_NSL = 3


def _tr_body(embp_hbm, embl_hbm, t2_v, stg_v, isems, osems):
    wid = lax.axis_index("s") * _NC + lax.axis_index("c")
    lanes = lax.iota(I32, 16)
    c15 = jnp.full((16,), 15, I32)
    tc0 = wid * _TPW

    def clamp(j):
        return jnp.minimum(tc0 + j * _G, jnp.int32(_TC - _G))

    def fire_in(g, s):
        for tr in range(2):
            pltpu.async_copy(
                embp_hbm.at[tr, pl.ds(g * 8, _G * 8)],
                t2_v.at[s, pl.ds(tr * _G * 8, _G * 8)],
                isems[s],
            )

    fire_in(clamp(0), 0)
    fire_in(clamp(1), 1)

    def pair(i, carry):
        for s in range(_NSL):
            j = i * _NSL + s
            g = clamp(j)

            @pl.when(j + 2 < _SPW)
            def _(j=j, s=s):
                fire_in(clamp(j + 2), (s + 2) % _NSL)

            for tr in range(2):
                pltpu.make_async_copy(
                    embp_hbm.at[tr, pl.ds(0, _G * 8)],
                    t2_v.at[s, pl.ds(tr * _G * 8, _G * 8)],
                    isems[s],
                ).wait()

            @pl.when(i > 0)
            def _(s=s):
                pltpu.make_async_copy(
                    stg_v.at[s], embl_hbm.at[pl.ds(0, _G * 128)], osems[s]
                ).wait()

            def tcl_body(tcl, carry2, s=s):
                rbase = tcl * 8
                sbase = tcl * 128
                for e0 in range(0, 128, 16):
                    ev = e0 + lanes
                    for k in range(_D):
                        dv = lax.bitwise_and(lanes + k, c15)
                        crow = (
                            lax.shift_right_logical(dv, 3) * (_G * 8)
                            + lax.bitwise_and(dv, jnp.full((16,), 7, I32))
                        )
                        v = plsc.load_gather(t2_v.at[s], [crow + rbase, ev])
                        plsc.store_scatter(stg_v.at[s], [sbase + ev, dv], v)
                return carry2

            lax.fori_loop(0, _G, tcl_body, 0)
            pltpu.async_copy(
                stg_v.at[s], embl_hbm.at[pl.ds(g * 128, _G * 128)], osems[s]
            )
        return carry

    lax.fori_loop(0, _SPW // _NSL, pair, 0)
    for s in range(_NSL):
        pltpu.make_async_copy(
            stg_v.at[s], embl_hbm.at[pl.ds(0, _G * 128)], osems[s]
        ).wait()


def kernel(x, phase, embedding, threshold):
    del phase
    xt = x.T.reshape(_F, _B // _SUB, _SUB)
    embp = (
        embedding.T.reshape(2, 8, _TC, 128)
        .transpose(0, 2, 1, 3)
        .reshape(2, _TC * 8, 128)
    )
    mesh = plsc.VectorSubcoreMesh(core_axis_name="c", subcore_axis_name="s")
    run_tr = pl.kernel(
        _tr_body,
        mesh=mesh,
        out_type=jax.ShapeDtypeStruct((_V, _D), F32),
        scratch_types=[
            pltpu.VMEM((_NSL, 2 * _G * 8, 128), F32),
            pltpu.VMEM((_NSL, _G * 128, 16), F32),
            [pltpu.SemaphoreType.DMA for _ in range(_NSL)],
            [pltpu.SemaphoreType.DMA for _ in range(_NSL)],
        ],
        compiler_params=pltpu.CompilerParams(
            needs_layout_passes=False, use_tc_tiling_on_sc=False
        ),
    )
    run = pl.kernel(
        _sc_body,
        mesh=mesh,
        out_type=jax.ShapeDtypeStruct((_F, 2, _B // _SUB, 8 * _SUB), F32),
        scratch_types=[
            pltpu.VMEM((_NBUF, _GPF, _SUB), I32),
            pltpu.VMEM((_NBUF, _PER_W, _D), F32),
            pltpu.VMEM((_NBUF, 2, _GPF, 8 * _SUB), F32),
            pltpu.VMEM((_F,), F32),
            [pltpu.SemaphoreType.DMA for _ in range(_NBUF)],
            [pltpu.SemaphoreType.DMA for _ in range(_NBUF)],
        ],
        compiler_params=pltpu.CompilerParams(
            needs_layout_passes=False, use_tc_tiling_on_sc=False
        ),
    )
    embl = run_tr(embp)
    out = run(xt, embl, threshold.reshape(-1))
    return (
        out.reshape(_F, 2, _B // _SUB, 8, _SUB)
        .transpose(2, 4, 0, 1, 3)
        .reshape(_B, _F, _D)
    )


# speedup vs baseline: 1.0283x; 1.0283x over previous
import jax
import jax.numpy as jnp
from jax import lax
from jax.experimental import pallas as pl
from jax.experimental.pallas import tpu as pltpu
from jax.experimental.pallas import tpu_sc as plsc

F32 = jnp.float32
I32 = jnp.int32

_V = 1040000
_B = 16384
_F = 26
_D = 16
_NC = 2
_NS = 16
_NW = _NC * _NS
_PER_W = _B // _NW
_SUB = 128
_GPF = _PER_W // _SUB
_BLK = _PER_W // 16
_NBUF = 3


def _sc_body(xt_hbm, emb_hbm, thr_hbm, out_hbm,
             idx_v, rows_v, trans_v, thr_v, gsems, osems):
    wid = lax.axis_index("s") * _NC + lax.axis_index("c")
    wb = wid * _PER_W
    tile0 = wid * _GPF
    pltpu.sync_copy(thr_hbm, thr_v)
    lanes = lax.iota(I32, 16)
    c15 = jnp.full((16,), 15, I32)

    def fire_gathers(f):
        s = f % _NBUF
        pltpu.sync_copy(xt_hbm.at[f, pl.ds(tile0, _GPF)], idx_v.at[s])
        return [
            pltpu.async_copy(
                emb_hbm.at[idx_v.at[s, j]],
                rows_v.at[s, pl.ds(j * _SUB, _SUB)],
                gsems[s],
            )
            for j in range(_GPF)
        ]

    pending_g = {0: fire_gathers(0)}
    pending_o = {}

    for f in range(_F):
        s = f % _NBUF
        if f + 1 < _F:
            if f - 2 >= 0:
                for cp in pending_o.pop(f - 2):
                    cp.wait()
            pending_g[f + 1] = fire_gathers(f + 1)
        for cp in pending_g.pop(f):
            cp.wait()

        rows = rows_v.at[s]
        trans = trans_v.at[s]
        thr = plsc.load_gather(thr_v, [jnp.full((16,), f, I32)])

        def blk(bi, _, rows=rows, trans=trans, thr=thr):
            rb = bi * 16
            ridx = rb + lanes
            tclv = jnp.zeros((16,), I32) + lax.shift_right_logical(rb, 7)
            kbase = lax.bitwise_and(rb, 127) + lanes
            acc = jnp.zeros((16,), F32)
            diags = []
            for d in range(_D):
                cidx = lax.bitwise_and(lanes + d, c15)
                v = plsc.load_gather(rows, [ridx, cidx])
                diags.append((cidx, v))
                acc = acc + jnp.abs(v)
            m = jnp.where(acc - thr > 0, jnp.float32(1.0), jnp.float32(0.0))
            for cidx, v in diags:
                trv = lax.shift_right_logical(cidx, 3)
                kv = lax.bitwise_and(cidx, jnp.full((16,), 7, I32)) * 128
                plsc.store_scatter(trans, [trv, tclv, kv + kbase], v * m)
            return 0

        lax.fori_loop(0, _BLK, blk, 0)
        pending_o[f] = [
            pltpu.async_copy(
                trans.at[tr2],
                out_hbm.at[f, tr2, pl.ds(_GPF * wid, _GPF)],
                osems[s],
            )
            for tr2 in range(2)
        ]

    for f in sorted(pending_o):
        for cp in pending_o[f]:
            cp.wait()


_TC = _V // 128
_TPW = 254


_G = 4
_SPW = 64


def _tr_body(embp_hbm, embl_hbm, t2_v, stg_v, isems, osems):
    wid = lax.axis_index("s") * _NC + lax.axis_index("c")
    lanes = lax.iota(I32, 16)
    c15 = jnp.full((16,), 15, I32)
    tc0 = wid * _TPW

    def clamp(j):
        return jnp.minimum(tc0 + j * _G, jnp.int32(_TC - _G))

    def fire_in(g, s):
        for tr in range(2):
            pltpu.async_copy(
                embp_hbm.at[tr, pl.ds(g * 8, _G * 8)],
                t2_v.at[s, pl.ds(tr * _G * 8, _G * 8)],
                isems[s],
            )

    fire_in(clamp(0), 0)

    def pair(i, carry):
        for s in range(2):
            j = i * 2 + s
            g = clamp(j)

            @pl.when(j + 1 < _SPW)
            def _(j=j, s=s):
                fire_in(clamp(j + 1), 1 - s)

            for tr in range(2):
                pltpu.make_async_copy(
                    embp_hbm.at[tr, pl.ds(0, _G * 8)],
                    t2_v.at[s, pl.ds(tr * _G * 8, _G * 8)],
                    isems[s],
                ).wait()

            @pl.when(i > 0)
            def _(s=s):
                pltpu.make_async_copy(
                    stg_v.at[s], embl_hbm.at[pl.ds(0, _G * 128)], osems[s]
                ).wait()

            def tcl_body(tcl, carry2, s=s):
                rbase = tcl * 8
                sbase = tcl * 128
                for e0 in range(0, 128, 16):
                    ev = e0 + lanes
                    for k in range(_D):
                        dv = lax.bitwise_and(lanes + k, c15)
                        crow = (
                            lax.shift_right_logical(dv, 3) * (_G * 8)
                            + lax.bitwise_and(dv, jnp.full((16,), 7, I32))
                        )
                        v = plsc.load_gather(t2_v.at[s], [crow + rbase, ev])
                        plsc.store_scatter(stg_v.at[s], [sbase + ev, dv], v)
                return carry2

            lax.fori_loop(0, _G, tcl_body, 0)
            pltpu.async_copy(
                stg_v.at[s], embl_hbm.at[pl.ds(g * 128, _G * 128)], osems[s]
            )
        return carry

    lax.fori_loop(0, _SPW // 2, pair, 0)
    for s in range(2):
        pltpu.make_async_copy(
            stg_v.at[s], embl_hbm.at[pl.ds(0, _G * 128)], osems[s]
        ).wait()


def kernel(x, phase, embedding, threshold):
    del phase
    xt = x.T.reshape(_F, _B // _SUB, _SUB)
    embp = (
        embedding.T.reshape(2, 8, _TC, 128)
        .transpose(0, 2, 1, 3)
        .reshape(2, _TC * 8, 128)
    )
    mesh = plsc.VectorSubcoreMesh(core_axis_name="c", subcore_axis_name="s")
    run_tr = pl.kernel(
        _tr_body,
        mesh=mesh,
        out_type=jax.ShapeDtypeStruct((_V, _D), F32),
        scratch_types=[
            pltpu.VMEM((2, 2 * _G * 8, 128), F32),
            pltpu.VMEM((2, _G * 128, 16), F32),
            [pltpu.SemaphoreType.DMA for _ in range(2)],
            [pltpu.SemaphoreType.DMA for _ in range(2)],
        ],
        compiler_params=pltpu.CompilerParams(
            needs_layout_passes=False, use_tc_tiling_on_sc=False
        ),
    )
    run = pl.kernel(
        _sc_body,
        mesh=mesh,
        out_type=jax.ShapeDtypeStruct((_F, 2, _B // _SUB, 8 * _SUB), F32),
        scratch_types=[
            pltpu.VMEM((_NBUF, _GPF, _SUB), I32),
            pltpu.VMEM((_NBUF, _PER_W, _D), F32),
            pltpu.VMEM((_NBUF, 2, _GPF, 8 * _SUB), F32),
            pltpu.VMEM((_F,), F32),
            [pltpu.SemaphoreType.DMA for _ in range(_NBUF)],
            [pltpu.SemaphoreType.DMA for _ in range(_NBUF)],
        ],
        compiler_params=pltpu.CompilerParams(
            needs_layout_passes=False, use_tc_tiling_on_sc=False
        ),
    )
    embl = run_tr(embp)
    out = run(xt, embl, threshold.reshape(-1))
    return (
        out.reshape(_F, 2, _B // _SUB, 8, _SUB)
        .transpose(2, 4, 0, 1, 3)
        .reshape(_B, _F, _D)
    )
